# Initial kernel scaffold; baseline (speedup 1.0000x reference)
import jax
import jax.numpy as jnp
from jax import lax
from jax.experimental import pallas as pl
from jax.experimental.pallas import tpu as pltpu
from jax.experimental.pallas import tpu_sc as plsc

NC = 2
NS = 16
NW = NC * NS
CHUNK = 128


def _mesh():
    return plsc.VectorSubcoreMesh(
        core_axis_name="c", subcore_axis_name="s", num_cores=NC, num_subcores=NS)


def _sc_gather(table, idx2):
    n_rows = idx2.shape[0]
    ch = n_rows // NW
    epw = ch * CHUNK
    feat = table.shape[1]

    def body(table_hbm, idx_hbm, out_hbm, idx_v, rows_v, sem):
        wid = lax.axis_index("c") * NS + lax.axis_index("s")
        rowbase = wid * ch
        base = wid * epw
        pltpu.sync_copy(idx_hbm.at[pl.ds(rowbase, ch)], idx_v)

        def fire(c, carry):
            pltpu.async_copy(table_hbm.at[idx_v.at[c]],
                             rows_v.at[pl.ds(c * CHUNK, CHUNK)], sem)
            return carry

        lax.fori_loop(0, ch, fire, 0)
        pltpu.make_async_copy(out_hbm.at[pl.ds(base, epw)], rows_v, sem).wait()
        pltpu.sync_copy(rows_v, out_hbm.at[pl.ds(base, epw)])

    return pl.kernel(
        body,
        out_type=jax.ShapeDtypeStruct((n_rows * CHUNK, feat), jnp.float32),
        mesh=_mesh(),
        scratch_types=[
            pltpu.VMEM((ch, CHUNK), jnp.int32),
            pltpu.VMEM((epw, feat), jnp.float32),
            pltpu.SemaphoreType.DMA,
        ],
    )(table, idx2)


def _sc_scatter(m, idx2, n_nodes):
    n_rows = idx2.shape[0]
    ch = n_rows // NW
    epw = ch * CHUNK
    feat = m.shape[1]
    rps = n_nodes // NS

    def body(m_hbm, idx_hbm, out_hbm, idx_v, m_v, zero_v, acc, sem):
        cid = lax.axis_index("c")
        sid = lax.axis_index("s")
        wid = cid * NS + sid
        rowbase = wid * ch
        base = wid * epw

        def zbody(i, carry):
            zero_v[i, :] = jnp.zeros((feat,), jnp.float32)
            return carry

        lax.fori_loop(0, rps, zbody, 0)
        pltpu.sync_copy(zero_v, acc.at[pl.ds(sid * rps, rps)])

        pltpu.sync_copy(idx_hbm.at[pl.ds(rowbase, ch)], idx_v)
        pltpu.sync_copy(m_hbm.at[pl.ds(base, epw)], m_v)
        plsc.subcore_barrier()

        def fire(c, carry):
            pltpu.async_copy(m_v.at[pl.ds(c * CHUNK, CHUNK)],
                             acc.at[idx_v.at[c]], sem, add=True)
            return carry

        lax.fori_loop(0, ch, fire, 0)
        pltpu.make_async_copy(m_hbm.at[pl.ds(base, epw)],
                              acc.at[pl.ds(0, epw)], sem).wait()
        plsc.subcore_barrier()
        pltpu.sync_copy(acc.at[pl.ds(sid * rps, rps)],
                        out_hbm.at[cid, pl.ds(sid * rps, rps)])

    return pl.kernel(
        body,
        out_type=jax.ShapeDtypeStruct((NC, n_nodes, feat), jnp.float32),
        mesh=_mesh(),
        scratch_types=[
            pltpu.VMEM((ch, CHUNK), jnp.int32),
            pltpu.VMEM((epw, feat), jnp.float32),
            pltpu.VMEM((rps, feat), jnp.float32),
            pltpu.VMEM_SHARED((n_nodes, feat), jnp.float32),
            pltpu.SemaphoreType.DMA,
        ],
    )(m, idx2)


def _tc_dense(ef_p, hs, Wm, bm, S1, S2, e_real, block):
    e_pad, nf = ef_p.shape
    grid = e_pad // block

    def body(ef_ref, hs_ref, w_ref, b_ref, s1_ref, s2_ref, m_ref):
        i = pl.program_id(0)
        ef = ef_ref[...]
        h = hs_ref[...]
        z = jnp.dot(ef, s1_ref[...], preferred_element_type=jnp.float32) * \
            jnp.dot(h, s2_ref[...], preferred_element_type=jnp.float32)
        m = jnp.dot(z, w_ref[...], preferred_element_type=jnp.float32) + \
            jnp.dot(h, b_ref[...], preferred_element_type=jnp.float32)
        rows = i * block + lax.broadcasted_iota(jnp.int32, (block, nf), 0)
        m_ref[...] = jnp.where(rows < e_real, m, 0.0)

    return pl.pallas_call(
        body,
        grid=(grid,),
        in_specs=[
            pl.BlockSpec((block, nf), lambda i: (i, 0)),
            pl.BlockSpec((block, nf), lambda i: (i, 0)),
            pl.BlockSpec((nf * nf, nf), lambda i: (0, 0)),
            pl.BlockSpec((nf, nf), lambda i: (0, 0)),
            pl.BlockSpec((nf, nf * nf), lambda i: (0, 0)),
            pl.BlockSpec((nf, nf * nf), lambda i: (0, 0)),
        ],
        out_specs=pl.BlockSpec((block, nf), lambda i: (i, 0)),
        out_shape=jax.ShapeDtypeStruct((e_pad, nf), jnp.float32),
    )(ef_p, hs, Wm, bm, S1, S2)


def _tc_post_bn(partials, nn_bias, gamma, beta):
    _, n, nf = partials.shape

    def body(p_ref, nb_ref, g_ref, be_ref, h_ref):
        agg = p_ref[0] + p_ref[1] + nb_ref[...]
        h = jnp.maximum(agg, 0.0)
        mean = jnp.mean(h, axis=0, keepdims=True)
        var = jnp.mean((h - mean) ** 2, axis=0, keepdims=True)
        h_ref[...] = g_ref[...] * (h - mean) * lax.rsqrt(var + 1e-5) + be_ref[...]

    return pl.pallas_call(
        body,
        out_shape=jax.ShapeDtypeStruct((n, nf), jnp.float32),
    )(partials, nn_bias.reshape(1, nf), gamma.reshape(1, nf), beta.reshape(1, nf))


def _tc_post_lsm(partials, nn_bias):
    _, n, nf = partials.shape

    def body(p_ref, nb_ref, o_ref):
        x = p_ref[0] + p_ref[1] + nb_ref[...]
        xm = x - jnp.max(x, axis=1, keepdims=True)
        o_ref[...] = xm - jnp.log(jnp.sum(jnp.exp(xm), axis=1, keepdims=True))

    return pl.pallas_call(
        body,
        out_shape=jax.ShapeDtypeStruct((n, nf), jnp.float32),
    )(partials, nn_bias.reshape(1, nf))


def kernel(inputs, edge_features, edge_index, W0, b0, nn_bias0, bn_gamma0,
           bn_beta0, W1, b1, nn_bias1):
    n, nf = inputs.shape
    e = edge_features.shape[0]
    src = edge_index[0]
    dst = edge_index[1]

    gran = NW * CHUNK
    e_pad = ((e + gran - 1) // gran) * gran
    pad = e_pad - e
    src_p = jnp.concatenate([src, jnp.zeros((pad,), jnp.int32)])
    dst_p = jnp.concatenate([dst, jnp.zeros((pad,), jnp.int32)])
    ef_p = jnp.concatenate([edge_features,
                            jnp.zeros((pad, nf), jnp.float32)], axis=0)
    src2 = src_p.reshape(e_pad // CHUNK, CHUNK)
    dst2 = dst_p.reshape(e_pad // CHUNK, CHUNK)

    eye = jnp.eye(nf, dtype=jnp.float32)
    S1 = jnp.repeat(eye, nf, axis=1)
    S2 = jnp.tile(eye, (1, nf))

    hs0 = _sc_gather(inputs, src2)
    m0 = _tc_dense(ef_p, hs0, W0.reshape(nf * nf, nf), b0.reshape(nf, nf),
                   S1, S2, e, 2048)
    p0 = _sc_scatter(m0, dst2, n)
    h = _tc_post_bn(p0, nn_bias0, bn_gamma0, bn_beta0)

    hs1 = _sc_gather(h, src2)
    m1 = _tc_dense(ef_p, hs1, W1.reshape(nf * nf, nf), b1.reshape(nf, nf),
                   S1, S2, e, 2048)
    p1 = _sc_scatter(m1, dst2, n)
    return _tc_post_lsm(p1, nn_bias1)


# speedup vs baseline: 3.2555x; 3.2555x over previous
import jax
import jax.numpy as jnp
from jax import lax
from jax.experimental import pallas as pl
from jax.experimental.pallas import tpu as pltpu
from jax.experimental.pallas import tpu_sc as plsc

NC = 2
NS = 16
NW = NC * NS
CHUNK = 128


def _mesh():
    return plsc.VectorSubcoreMesh(
        core_axis_name="c", subcore_axis_name="s", num_cores=NC, num_subcores=NS)


def _sc_gather(table, idx2):
    n_rows = idx2.shape[0]
    ch = n_rows // NW
    epw = ch * CHUNK
    feat = table.shape[1]

    kk = 8

    def body(table_hbm, idx_hbm, out_hbm, idx_v, rows_v, sem):
        wid = lax.axis_index("c") * NS + lax.axis_index("s")
        rowbase = wid * ch
        base = wid * epw
        pltpu.sync_copy(idx_hbm.at[pl.ds(rowbase, ch)], idx_v)

        def group(g, carry):
            c0 = g * kk
            handles = [
                pltpu.async_copy(table_hbm.at[idx_v.at[c0 + j]],
                                 rows_v.at[pl.ds((c0 + j) * CHUNK, CHUNK)], sem)
                for j in range(kk)
            ]
            for hd in handles:
                hd.wait()
            return carry

        lax.fori_loop(0, ch // kk, group, 0)
        pltpu.sync_copy(rows_v, out_hbm.at[pl.ds(base, epw)])

    return pl.kernel(
        body,
        out_type=jax.ShapeDtypeStruct((n_rows * CHUNK, feat), jnp.float32),
        mesh=_mesh(),
        compiler_params=pltpu.CompilerParams(use_tc_tiling_on_sc=False),
        scratch_types=[
            pltpu.VMEM((ch, CHUNK), jnp.int32),
            pltpu.VMEM((epw, feat), jnp.float32),
            pltpu.SemaphoreType.DMA,
        ],
    )(table, idx2)


def _sc_scatter(m, idx2, n_nodes):
    n_rows = idx2.shape[0]
    ch = n_rows // NW
    epw = ch * CHUNK
    feat = m.shape[1]
    rps = n_nodes // NS
    kk = 8

    def body(m_hbm, idx_hbm, out_hbm, idx_v, m_v, zero_v, acc, sem):
        cid = lax.axis_index("c")
        sid = lax.axis_index("s")
        wid = cid * NS + sid
        rowbase = wid * ch
        base = wid * epw

        def zbody(i, carry):
            zero_v[i, :] = jnp.zeros((feat,), jnp.float32)
            return carry

        lax.fori_loop(0, rps, zbody, 0)
        pltpu.sync_copy(zero_v, acc.at[pl.ds(sid * rps, rps)])

        pltpu.sync_copy(idx_hbm.at[pl.ds(rowbase, ch)], idx_v)
        pltpu.sync_copy(m_hbm.at[pl.ds(base, epw)], m_v)
        plsc.subcore_barrier()

        def group(g, carry):
            c0 = g * kk
            handles = [
                pltpu.async_copy(m_v.at[pl.ds((c0 + j) * CHUNK, CHUNK)],
                                 acc.at[idx_v.at[c0 + j]], sem, add=True)
                for j in range(kk)
            ]
            for hd in handles:
                hd.wait()
            return carry

        lax.fori_loop(0, ch // kk, group, 0)
        plsc.subcore_barrier()
        pltpu.sync_copy(acc.at[pl.ds(sid * rps, rps)],
                        out_hbm.at[pl.ds(cid * n_nodes + sid * rps, rps)])

    return pl.kernel(
        body,
        out_type=jax.ShapeDtypeStruct((NC * n_nodes, feat), jnp.float32),
        mesh=_mesh(),
        compiler_params=pltpu.CompilerParams(use_tc_tiling_on_sc=False),
        scratch_types=[
            pltpu.VMEM((ch, CHUNK), jnp.int32),
            pltpu.VMEM((epw, feat), jnp.float32),
            pltpu.VMEM((rps, feat), jnp.float32),
            pltpu.VMEM_SHARED((n_nodes, feat), jnp.float32),
            pltpu.SemaphoreType.DMA,
        ],
    )(m, idx2)


def _tc_dense(ef_p, hs, Wm, bm, S1, S2, e_real, block):
    e_pad, nf = ef_p.shape
    grid = e_pad // block

    def body(ef_ref, hs_ref, w_ref, b_ref, s1_ref, s2_ref, m_ref):
        i = pl.program_id(0)
        ef = ef_ref[...]
        h = hs_ref[...]
        z = jnp.dot(ef, s1_ref[...], preferred_element_type=jnp.float32) * \
            jnp.dot(h, s2_ref[...], preferred_element_type=jnp.float32)
        m = jnp.dot(z, w_ref[...], preferred_element_type=jnp.float32) + \
            jnp.dot(h, b_ref[...], preferred_element_type=jnp.float32)
        rows = i * block + lax.broadcasted_iota(jnp.int32, (block, nf), 0)
        m_ref[...] = jnp.where(rows < e_real, m, 0.0)

    return pl.pallas_call(
        body,
        grid=(grid,),
        in_specs=[
            pl.BlockSpec((block, nf), lambda i: (i, 0)),
            pl.BlockSpec((block, nf), lambda i: (i, 0)),
            pl.BlockSpec((nf * nf, nf), lambda i: (0, 0)),
            pl.BlockSpec((nf, nf), lambda i: (0, 0)),
            pl.BlockSpec((nf, nf * nf), lambda i: (0, 0)),
            pl.BlockSpec((nf, nf * nf), lambda i: (0, 0)),
        ],
        out_specs=pl.BlockSpec((block, nf), lambda i: (i, 0)),
        out_shape=jax.ShapeDtypeStruct((e_pad, nf), jnp.float32),
    )(ef_p, hs, Wm, bm, S1, S2)


def _tc_post_bn(partials, nn_bias, gamma, beta):
    _, n, nf = partials.shape

    def body(p_ref, nb_ref, g_ref, be_ref, h_ref):
        agg = p_ref[0] + p_ref[1] + nb_ref[...]
        h = jnp.maximum(agg, 0.0)
        mean = jnp.mean(h, axis=0, keepdims=True)
        var = jnp.mean((h - mean) ** 2, axis=0, keepdims=True)
        h_ref[...] = g_ref[...] * (h - mean) * lax.rsqrt(var + 1e-5) + be_ref[...]

    return pl.pallas_call(
        body,
        out_shape=jax.ShapeDtypeStruct((n, nf), jnp.float32),
    )(partials, nn_bias.reshape(1, nf), gamma.reshape(1, nf), beta.reshape(1, nf))


def _tc_post_lsm(partials, nn_bias):
    _, n, nf = partials.shape

    def body(p_ref, nb_ref, o_ref):
        x = p_ref[0] + p_ref[1] + nb_ref[...]
        xm = x - jnp.max(x, axis=1, keepdims=True)
        o_ref[...] = xm - jnp.log(jnp.sum(jnp.exp(xm), axis=1, keepdims=True))

    return pl.pallas_call(
        body,
        out_shape=jax.ShapeDtypeStruct((n, nf), jnp.float32),
    )(partials, nn_bias.reshape(1, nf))


def kernel(inputs, edge_features, edge_index, W0, b0, nn_bias0, bn_gamma0,
           bn_beta0, W1, b1, nn_bias1):
    n, nf = inputs.shape
    e = edge_features.shape[0]
    src = edge_index[0]
    dst = edge_index[1]

    gran = NW * CHUNK
    e_pad = ((e + gran - 1) // gran) * gran
    pad = e_pad - e
    src_p = jnp.concatenate([src, jnp.zeros((pad,), jnp.int32)])
    dst_p = jnp.concatenate([dst, jnp.zeros((pad,), jnp.int32)])
    ef_p = jnp.concatenate([edge_features,
                            jnp.zeros((pad, nf), jnp.float32)], axis=0)
    src2 = src_p.reshape(e_pad // CHUNK, CHUNK)
    dst2 = dst_p.reshape(e_pad // CHUNK, CHUNK)

    eye = jnp.eye(nf, dtype=jnp.float32)
    S1 = jnp.repeat(eye, nf, axis=1)
    S2 = jnp.tile(eye, (1, nf))

    hs0 = _sc_gather(inputs, src2)
    m0 = _tc_dense(ef_p, hs0, W0.reshape(nf * nf, nf), b0.reshape(nf, nf),
                   S1, S2, e, 2048)
    p0 = _sc_scatter(m0, dst2, n).reshape(NC, n, nf)
    h = _tc_post_bn(p0, nn_bias0, bn_gamma0, bn_beta0)

    hs1 = _sc_gather(h, src2)
    m1 = _tc_dense(ef_p, hs1, W1.reshape(nf * nf, nf), b1.reshape(nf, nf),
                   S1, S2, e, 2048)
    p1 = _sc_scatter(m1, dst2, n).reshape(NC, n, nf)
    return _tc_post_lsm(p1, nn_bias1)


# speedup vs baseline: 3.9302x; 1.2072x over previous
import jax
import jax.numpy as jnp
from jax import lax
from jax.experimental import pallas as pl
from jax.experimental.pallas import tpu as pltpu
from jax.experimental.pallas import tpu_sc as plsc

NC = 2
NS = 16
NW = NC * NS
CHUNK = 125


def _mesh():
    return plsc.VectorSubcoreMesh(
        core_axis_name="c", subcore_axis_name="s", num_cores=NC, num_subcores=NS)


def _sc_gather(table, idx2):
    n_rows = idx2.shape[0]
    ch = n_rows // NW
    epw = ch * CHUNK
    feat = table.shape[1]

    kk = 8

    def body(table_hbm, idx_hbm, out_hbm, idx_v, rows_v, sem):
        wid = lax.axis_index("c") * NS + lax.axis_index("s")
        rowbase = wid * ch
        base = wid * epw
        pltpu.sync_copy(idx_hbm.at[pl.ds(rowbase, ch)], idx_v)

        def group(g, carry):
            c0 = g * kk
            handles = [
                pltpu.async_copy(table_hbm.at[idx_v.at[c0 + j]],
                                 rows_v.at[pl.ds((c0 + j) * CHUNK, CHUNK)], sem)
                for j in range(kk)
            ]
            for hd in handles:
                hd.wait()
            return carry

        lax.fori_loop(0, ch // kk, group, 0)
        pltpu.sync_copy(rows_v, out_hbm.at[pl.ds(base, epw)])

    return pl.kernel(
        body,
        out_type=jax.ShapeDtypeStruct((n_rows * CHUNK, feat), jnp.float32),
        mesh=_mesh(),
        compiler_params=pltpu.CompilerParams(use_tc_tiling_on_sc=False),
        scratch_types=[
            pltpu.VMEM((ch, CHUNK), jnp.int32),
            pltpu.VMEM((epw, feat), jnp.float32),
            pltpu.SemaphoreType.DMA,
        ],
    )(table, idx2)


def _sc_scatter(m, idx2, n_nodes):
    n_rows = idx2.shape[0]
    ch = n_rows // NW
    epw = ch * CHUNK
    feat = m.shape[1]
    rps = n_nodes // NS
    kk = 8

    def body(m_hbm, idx_hbm, out_hbm, idx_v, m_v, zero_v, acc, sem):
        cid = lax.axis_index("c")
        sid = lax.axis_index("s")
        wid = cid * NS + sid
        rowbase = wid * ch
        base = wid * epw

        def zbody(i, carry):
            zero_v[i, :] = jnp.zeros((feat,), jnp.float32)
            return carry

        lax.fori_loop(0, rps, zbody, 0)
        pltpu.sync_copy(zero_v, acc.at[pl.ds(sid * rps, rps)])

        pltpu.sync_copy(idx_hbm.at[pl.ds(rowbase, ch)], idx_v)
        pltpu.sync_copy(m_hbm.at[pl.ds(base, epw)], m_v)
        plsc.subcore_barrier()

        def group(g, carry):
            c0 = g * kk
            handles = [
                pltpu.async_copy(m_v.at[pl.ds((c0 + j) * CHUNK, CHUNK)],
                                 acc.at[idx_v.at[c0 + j]], sem, add=True)
                for j in range(kk)
            ]
            for hd in handles:
                hd.wait()
            return carry

        lax.fori_loop(0, ch // kk, group, 0)
        plsc.subcore_barrier()
        pltpu.sync_copy(acc.at[pl.ds(sid * rps, rps)],
                        out_hbm.at[pl.ds(cid * n_nodes + sid * rps, rps)])

    return pl.kernel(
        body,
        out_type=jax.ShapeDtypeStruct((NC * n_nodes, feat), jnp.float32),
        mesh=_mesh(),
        compiler_params=pltpu.CompilerParams(use_tc_tiling_on_sc=False),
        scratch_types=[
            pltpu.VMEM((ch, CHUNK), jnp.int32),
            pltpu.VMEM((epw, feat), jnp.float32),
            pltpu.VMEM((rps, feat), jnp.float32),
            pltpu.VMEM_SHARED((n_nodes, feat), jnp.float32),
            pltpu.SemaphoreType.DMA,
        ],
    )(m, idx2)


def _tc_dense(ef_p, hs, Wm, bm, S1, S2, e_real, block):
    e_pad, nf = ef_p.shape
    grid = e_pad // block

    def body(ef_ref, hs_ref, w_ref, b_ref, s1_ref, s2_ref, m_ref):
        i = pl.program_id(0)
        ef = ef_ref[...]
        h = hs_ref[...]
        z = jnp.dot(ef, s1_ref[...], preferred_element_type=jnp.float32) * \
            jnp.dot(h, s2_ref[...], preferred_element_type=jnp.float32)
        m = jnp.dot(z, w_ref[...], preferred_element_type=jnp.float32) + \
            jnp.dot(h, b_ref[...], preferred_element_type=jnp.float32)
        if e_real != e_pad:
            rows = i * block + lax.broadcasted_iota(jnp.int32, (block, nf), 0)
            m = jnp.where(rows < e_real, m, 0.0)
        m_ref[...] = m

    return pl.pallas_call(
        body,
        grid=(grid,),
        in_specs=[
            pl.BlockSpec((block, nf), lambda i: (i, 0)),
            pl.BlockSpec((block, nf), lambda i: (i, 0)),
            pl.BlockSpec((nf * nf, nf), lambda i: (0, 0)),
            pl.BlockSpec((nf, nf), lambda i: (0, 0)),
            pl.BlockSpec((nf, nf * nf), lambda i: (0, 0)),
            pl.BlockSpec((nf, nf * nf), lambda i: (0, 0)),
        ],
        out_specs=pl.BlockSpec((block, nf), lambda i: (i, 0)),
        out_shape=jax.ShapeDtypeStruct((e_pad, nf), jnp.float32),
    )(ef_p, hs, Wm, bm, S1, S2)


def _tc_post_bn(partials, nn_bias, gamma, beta):
    _, n, nf = partials.shape

    def body(p_ref, nb_ref, g_ref, be_ref, h_ref):
        agg = p_ref[0] + p_ref[1] + nb_ref[...]
        h = jnp.maximum(agg, 0.0)
        mean = jnp.mean(h, axis=0, keepdims=True)
        var = jnp.mean((h - mean) ** 2, axis=0, keepdims=True)
        h_ref[...] = g_ref[...] * (h - mean) * lax.rsqrt(var + 1e-5) + be_ref[...]

    return pl.pallas_call(
        body,
        out_shape=jax.ShapeDtypeStruct((n, nf), jnp.float32),
    )(partials, nn_bias.reshape(1, nf), gamma.reshape(1, nf), beta.reshape(1, nf))


def _tc_post_lsm(partials, nn_bias):
    _, n, nf = partials.shape

    def body(p_ref, nb_ref, o_ref):
        x = p_ref[0] + p_ref[1] + nb_ref[...]
        xm = x - jnp.max(x, axis=1, keepdims=True)
        o_ref[...] = xm - jnp.log(jnp.sum(jnp.exp(xm), axis=1, keepdims=True))

    return pl.pallas_call(
        body,
        out_shape=jax.ShapeDtypeStruct((n, nf), jnp.float32),
    )(partials, nn_bias.reshape(1, nf))


def kernel(inputs, edge_features, edge_index, W0, b0, nn_bias0, bn_gamma0,
           bn_beta0, W1, b1, nn_bias1):
    n, nf = inputs.shape
    e = edge_features.shape[0]
    src = edge_index[0]
    dst = edge_index[1]

    gran = NW * CHUNK
    e_pad = ((e + gran - 1) // gran) * gran
    pad = e_pad - e
    if pad:
        src_p = jnp.concatenate([src, jnp.zeros((pad,), jnp.int32)])
        dst_p = jnp.concatenate([dst, jnp.zeros((pad,), jnp.int32)])
        ef_p = jnp.concatenate([edge_features,
                                jnp.zeros((pad, nf), jnp.float32)], axis=0)
    else:
        src_p, dst_p, ef_p = src, dst, edge_features
    src2 = src_p.reshape(e_pad // CHUNK, CHUNK)
    dst2 = dst_p.reshape(e_pad // CHUNK, CHUNK)

    eye = jnp.eye(nf, dtype=jnp.float32)
    S1 = jnp.repeat(eye, nf, axis=1)
    S2 = jnp.tile(eye, (1, nf))

    hs0 = _sc_gather(inputs, src2)
    m0 = _tc_dense(ef_p, hs0, W0.reshape(nf * nf, nf), b0.reshape(nf, nf),
                   S1, S2, e, 4000)
    p0 = _sc_scatter(m0, dst2, n).reshape(NC, n, nf)
    h = _tc_post_bn(p0, nn_bias0, bn_gamma0, bn_beta0)

    hs1 = _sc_gather(h, src2)
    m1 = _tc_dense(ef_p, hs1, W1.reshape(nf * nf, nf), b1.reshape(nf, nf),
                   S1, S2, e, 4000)
    p1 = _sc_scatter(m1, dst2, n).reshape(NC, n, nf)
    return _tc_post_lsm(p1, nn_bias1)
